# baseline (speedup 1.0000x reference)
import functools

import jax
import jax.numpy as jnp
from jax import lax
from jax.experimental import pallas as pl
from jax.experimental.pallas import tpu as pltpu
from jax.experimental.pallas import tpu_sc as plsc

N = 10000
D = 128
NC = 2
NS = 16
NW = NC * NS
CHUNK = 128
N_R = 10112
STRIPE = N_R // NS


def _sc_degree(dst3, z128, ones128, n_chunks):
    mesh = plsc.VectorSubcoreMesh(core_axis_name="c", subcore_axis_name="s")

    @functools.partial(
        pl.kernel,
        mesh=mesh,
        out_type=jax.ShapeDtypeStruct((NC, N_R, D), jnp.float32),
        scratch_types=[
            pltpu.VMEM((n_chunks, CHUNK), jnp.int32),
            pltpu.VMEM((CHUNK, D), jnp.float32),
            pltpu.VMEM_SHARED((N_R, D), jnp.float32),
        ],
    )
    def deg_k(dst_hbm, z128_hbm, ones_hbm, d_hbm, dst_v, ones_v, deg_s):
        c = lax.axis_index("c")
        s = lax.axis_index("s")
        wid = c * NS + s
        pltpu.sync_copy(dst_hbm.at[wid], dst_v)
        pltpu.sync_copy(ones_hbm, ones_v)
        row0 = s * STRIPE
        pltpu.sync_copy(z128_hbm, deg_s.at[pl.ds(row0, STRIPE)])
        plsc.subcore_barrier()

        def body(j, carry):
            pltpu.sync_copy(ones_v, deg_s.at[dst_v.at[j]], add=True)
            return carry

        lax.fori_loop(0, n_chunks, body, 0)
        plsc.subcore_barrier()
        pltpu.sync_copy(deg_s.at[pl.ds(row0, STRIPE)],
                        d_hbm.at[c].at[pl.ds(row0, STRIPE)])

    return deg_k(dst3, z128, ones128)


def _sc_aggregate(h, src3, dst3, z128, n_chunks):
    mesh = plsc.VectorSubcoreMesh(core_axis_name="c", subcore_axis_name="s")
    half = n_chunks // 2

    @functools.partial(
        pl.kernel,
        mesh=mesh,
        out_type=jax.ShapeDtypeStruct((NC, N_R, D), jnp.float32),
        scratch_types=[
            pltpu.VMEM((half, CHUNK), jnp.int32),
            pltpu.VMEM((half, CHUNK), jnp.int32),
            pltpu.VMEM((CHUNK, D), jnp.float32),
            pltpu.VMEM((CHUNK, D), jnp.float32),
            pltpu.VMEM_SHARED((N_R, D), jnp.float32),
            pltpu.SemaphoreType.DMA,
            pltpu.SemaphoreType.DMA,
        ],
    )
    def agg(h_hbm, src_hbm, dst_hbm, z128_hbm, p_hbm,
            src_v, dst_v, buf_a, buf_b, acc_s, sga, sgb):
        c = lax.axis_index("c")
        s = lax.axis_index("s")
        wid = c * NS + s
        row0 = s * STRIPE
        pltpu.sync_copy(z128_hbm, acc_s.at[pl.ds(row0, STRIPE)])
        plsc.subcore_barrier()

        def gath(buf, sem, j):
            pltpu.async_copy(h_hbm.at[src_v.at[j]], buf, sem)

        def gath_wait(buf, sem, j):
            pltpu.make_async_copy(h_hbm.at[src_v.at[j]], buf, sem).wait()

        def scat(buf, j):
            pltpu.sync_copy(buf, acc_s.at[dst_v.at[j]], add=True)

        for hp in range(2):
            pltpu.sync_copy(src_hbm.at[wid].at[pl.ds(hp * half, half)],
                            src_v)
            pltpu.sync_copy(dst_hbm.at[wid].at[pl.ds(hp * half, half)],
                            dst_v)
            gath(buf_a, sga, 0)

            def body(i, carry):
                ja = 2 * i
                jb = 2 * i + 1
                gath(buf_b, sgb, jb)
                gath_wait(buf_a, sga, ja)
                scat(buf_a, ja)

                @pl.when(jb + 1 < half)
                def _():
                    gath(buf_a, sga, ja + 2)

                gath_wait(buf_b, sgb, jb)
                scat(buf_b, jb)
                return carry

            lax.fori_loop(0, half // 2, body, 0)
        plsc.subcore_barrier()
        pltpu.sync_copy(acc_s.at[pl.ds(row0, STRIPE)],
                        p_hbm.at[c].at[pl.ds(row0, STRIPE)])

    return agg(h, src3, dst3, z128)


BLK = 400


def _tc_layer(p, d, h, Wl, bl, Wr, relu):
    nb = N // BLK

    def body(p_ref, d_ref, h_ref, wl_ref, bl_ref, wr_ref, o_ref):
        deg = d_ref[0, :, 0:1] + d_ref[1, :, 0:1]
        mean = (p_ref[0] + p_ref[1]) / jnp.maximum(deg, 1.0)
        out = (lax.dot_general(mean, wl_ref[...], (((1,), (1,)), ((), ())),
                               preferred_element_type=jnp.float32)
               + lax.dot_general(h_ref[...], wr_ref[...],
                                 (((1,), (1,)), ((), ())),
                                 preferred_element_type=jnp.float32)
               + bl_ref[...])
        if relu:
            out = jnp.maximum(out, 0.0)
        o_ref[...] = out

    return pl.pallas_call(
        body,
        grid=(nb,),
        in_specs=[
            pl.BlockSpec((NC, BLK, D), lambda i: (0, i, 0)),
            pl.BlockSpec((NC, BLK, D), lambda i: (0, i, 0)),
            pl.BlockSpec((BLK, D), lambda i: (i, 0)),
            pl.BlockSpec((D, D), lambda i: (0, 0)),
            pl.BlockSpec((1, D), lambda i: (0, 0)),
            pl.BlockSpec((D, D), lambda i: (0, 0)),
        ],
        out_specs=pl.BlockSpec((BLK, D), lambda i: (i, 0)),
        out_shape=jax.ShapeDtypeStruct((N, D), jnp.float32),
    )(p, d, h, Wl, bl, Wr)


def kernel(x, edge_index, Wl1, bl1, Wr1, Wl2, bl2, Wr2, Wl3, bl3, Wr3):
    src = edge_index[0]
    dst = edge_index[1]
    e = src.shape[0]
    n_chunks = -(-e // (NW * CHUNK))
    n_chunks = -(-n_chunks // 4) * 4
    e_pad = NW * CHUNK * n_chunks
    pad = e_pad - e
    src_p = jnp.concatenate(
        [src, jnp.zeros((pad,), jnp.int32)]).reshape(NW, n_chunks, CHUNK)
    dst_p = jnp.concatenate(
        [dst, jnp.full((pad,), N, jnp.int32)]).reshape(NW, n_chunks, CHUNK)
    z128 = jnp.zeros((STRIPE, D), jnp.float32)
    ones128 = jnp.ones((CHUNK, D), jnp.float32)

    dg = _sc_degree(dst_p, z128, ones128, n_chunks)[:, :N]

    def layer(h, Wl, bl, Wr, relu):
        p = _sc_aggregate(h, src_p, dst_p, z128, n_chunks)
        return _tc_layer(p[:, :N], dg, h, Wl, bl.reshape(1, D), Wr, relu)

    h = layer(x, Wl1, bl1, Wr1, True)
    h = layer(h, Wl2, bl2, Wr2, True)
    return layer(h, Wl3, bl3, Wr3, False)


# speedup vs baseline: 1.8181x; 1.8181x over previous
import functools

import jax
import jax.numpy as jnp
from jax import lax
from jax.experimental import pallas as pl
from jax.experimental.pallas import tpu as pltpu
from jax.experimental.pallas import tpu_sc as plsc

N = 10000
D = 128
NC = 2
NS = 16
NW = NC * NS
CHUNK = 128
N_R = 10112
STRIPE = N_R // NS
SPLIT_A = 1
SPLIT_B = 1


def _sc_degree(dst3, z128, ones128, m, na, nb):
    mesh = plsc.VectorSubcoreMesh(core_axis_name="c", subcore_axis_name="s")

    @functools.partial(
        pl.kernel,
        mesh=mesh,
        out_type=jax.ShapeDtypeStruct((NC, N_R, D), jnp.float32),
        scratch_types=[
            pltpu.VMEM((m, CHUNK), jnp.int32),
            pltpu.VMEM((CHUNK, D), jnp.float32),
            pltpu.VMEM_SHARED((N_R, D), jnp.float32),
        ],
    )
    def deg_k(dst_hbm, z128_hbm, ones_hbm, d_hbm, dst_v, ones_v, deg_s):
        c = lax.axis_index("c")
        s = lax.axis_index("s")
        wid = c * NS + s
        n_loc = jnp.where(c == 0, na, nb)
        pltpu.sync_copy(dst_hbm.at[wid], dst_v)
        pltpu.sync_copy(ones_hbm, ones_v)
        row0 = s * STRIPE
        pltpu.sync_copy(z128_hbm, deg_s.at[pl.ds(row0, STRIPE)])
        plsc.subcore_barrier()

        def body(j, carry):
            pltpu.sync_copy(ones_v, deg_s.at[dst_v.at[j]], add=True)
            return carry

        lax.fori_loop(0, n_loc, body, 0)
        plsc.subcore_barrier()
        pltpu.sync_copy(deg_s.at[pl.ds(row0, STRIPE)],
                        d_hbm.at[c].at[pl.ds(row0, STRIPE)])

    return deg_k(dst3, z128, ones128)


def _sc_aggregate(h, src3, dst3, z128, m, na, nb):
    mesh = plsc.VectorSubcoreMesh(core_axis_name="c", subcore_axis_name="s")

    @functools.partial(
        pl.kernel,
        mesh=mesh,
        out_type=jax.ShapeDtypeStruct((NC, N_R, D), jnp.float32),
        scratch_types=[
            pltpu.VMEM((m, CHUNK), jnp.int32),
            pltpu.VMEM((m, CHUNK), jnp.int32),
            pltpu.VMEM((CHUNK, D), jnp.float32),
            pltpu.VMEM_SHARED((N_R, D), jnp.float32),
        ],
    )
    def agg(h_hbm, src_hbm, dst_hbm, z128_hbm, p_hbm,
            src_v, dst_v, rows_v, acc_s):
        c = lax.axis_index("c")
        s = lax.axis_index("s")
        wid = c * NS + s
        n_loc = jnp.where(c == 0, na, nb)
        pltpu.sync_copy(src_hbm.at[wid], src_v)
        pltpu.sync_copy(dst_hbm.at[wid], dst_v)
        row0 = s * STRIPE
        pltpu.sync_copy(z128_hbm, acc_s.at[pl.ds(row0, STRIPE)])
        plsc.subcore_barrier()

        def body(j, carry):
            pltpu.sync_copy(h_hbm.at[src_v.at[j]], rows_v)
            pltpu.sync_copy(rows_v, acc_s.at[dst_v.at[j]], add=True)
            return carry

        lax.fori_loop(0, n_loc, body, 0)
        plsc.subcore_barrier()
        pltpu.sync_copy(acc_s.at[pl.ds(row0, STRIPE)],
                        p_hbm.at[c].at[pl.ds(row0, STRIPE)])

    return agg(h, src3, dst3, z128)


BLK = 400


def _tc_layer(p, d, h, Wl, bl, Wr, relu):
    nb = N // BLK

    def body(p_ref, d_ref, h_ref, wl_ref, bl_ref, wr_ref, o_ref):
        deg = d_ref[0, :, 0:1] + d_ref[1, :, 0:1]
        mean = (p_ref[0] + p_ref[1]) / jnp.maximum(deg, 1.0)
        out = (lax.dot_general(mean, wl_ref[...], (((1,), (1,)), ((), ())),
                               preferred_element_type=jnp.float32)
               + lax.dot_general(h_ref[...], wr_ref[...],
                                 (((1,), (1,)), ((), ())),
                                 preferred_element_type=jnp.float32)
               + bl_ref[...])
        if relu:
            out = jnp.maximum(out, 0.0)
        o_ref[...] = out

    return pl.pallas_call(
        body,
        grid=(nb,),
        in_specs=[
            pl.BlockSpec((NC, BLK, D), lambda i: (0, i, 0)),
            pl.BlockSpec((NC, BLK, D), lambda i: (0, i, 0)),
            pl.BlockSpec((BLK, D), lambda i: (i, 0)),
            pl.BlockSpec((D, D), lambda i: (0, 0)),
            pl.BlockSpec((1, D), lambda i: (0, 0)),
            pl.BlockSpec((D, D), lambda i: (0, 0)),
        ],
        out_specs=pl.BlockSpec((BLK, D), lambda i: (i, 0)),
        out_shape=jax.ShapeDtypeStruct((N, D), jnp.float32),
    )(p, d, h, Wl, bl, Wr)


def kernel(x, edge_index, Wl1, bl1, Wr1, Wl2, bl2, Wr2, Wl3, bl3, Wr3):
    src = edge_index[0]
    dst = edge_index[1]
    e = src.shape[0]
    n_chunks = -(-e // (NS * CHUNK))
    na = (n_chunks * SPLIT_A + SPLIT_A + SPLIT_B - 1) // (SPLIT_A + SPLIT_B)
    nb = n_chunks - na
    if nb < 1:
        nb = 1
    m = max(na, nb)
    e_pad = NS * CHUNK * (na + nb)
    pad = e_pad - e
    sinkv = jnp.full((pad,), N, jnp.int32)
    src_p = jnp.concatenate([src, jnp.zeros((pad,), jnp.int32)])
    dst_p = jnp.concatenate([dst, sinkv])

    def part(v):
        p0 = v[:NS * na * CHUNK].reshape(NS, na, CHUNK)
        p1 = v[NS * na * CHUNK:].reshape(NS, nb, CHUNK)
        f0 = jnp.concatenate(
            [p0, jnp.full((NS, m - na, CHUNK), N, jnp.int32)], axis=1)
        f1 = jnp.concatenate(
            [p1, jnp.full((NS, m - nb, CHUNK), N, jnp.int32)], axis=1)
        return jnp.concatenate([f0, f1], axis=0)

    src_p = part(jnp.where(jnp.arange(e_pad) < e, src_p, 0))
    dst_p = part(dst_p)
    z128 = jnp.zeros((STRIPE, D), jnp.float32)
    ones128 = jnp.ones((CHUNK, D), jnp.float32)

    dg = _sc_degree(dst_p, z128, ones128, m, na, nb)[:, :N]

    def layer(h, Wl, bl, Wr, relu):
        p = _sc_aggregate(h, src_p, dst_p, z128, m, na, nb)
        return _tc_layer(p[:, :N], dg, h, Wl, bl.reshape(1, D), Wr, relu)

    h = layer(x, Wl1, bl1, Wr1, True)
    h = layer(h, Wl2, bl2, Wr2, True)
    return layer(h, Wl3, bl3, Wr3, False)


# speedup vs baseline: 1.9345x; 1.0640x over previous
import functools

import jax
import jax.numpy as jnp
from jax import lax
from jax.experimental import pallas as pl
from jax.experimental.pallas import tpu as pltpu
from jax.experimental.pallas import tpu_sc as plsc

N = 10000
D = 128
NC = 2
NS = 16
NW = NC * NS
CHUNK = 128
N_R = 10112
STRIPE = N_R // NS
SPLIT_A = 3
SPLIT_B = 2


def _sc_degree(dst3, z128, ones128, m, na, nb):
    mesh = plsc.VectorSubcoreMesh(core_axis_name="c", subcore_axis_name="s")

    @functools.partial(
        pl.kernel,
        mesh=mesh,
        out_type=jax.ShapeDtypeStruct((NC, N_R, D), jnp.float32),
        scratch_types=[
            pltpu.VMEM((m, CHUNK), jnp.int32),
            pltpu.VMEM((CHUNK, D), jnp.float32),
            pltpu.VMEM_SHARED((N_R, D), jnp.float32),
        ],
    )
    def deg_k(dst_hbm, z128_hbm, ones_hbm, d_hbm, dst_v, ones_v, deg_s):
        c = lax.axis_index("c")
        s = lax.axis_index("s")
        wid = c * NS + s
        n_loc = jnp.where(c == 0, na, nb)
        pltpu.sync_copy(dst_hbm.at[wid], dst_v)
        pltpu.sync_copy(ones_hbm, ones_v)
        row0 = s * STRIPE
        pltpu.sync_copy(z128_hbm, deg_s.at[pl.ds(row0, STRIPE)])
        plsc.subcore_barrier()

        def body(j, carry):
            pltpu.sync_copy(ones_v, deg_s.at[dst_v.at[j]], add=True)
            return carry

        lax.fori_loop(0, n_loc, body, 0)
        plsc.subcore_barrier()
        pltpu.sync_copy(deg_s.at[pl.ds(row0, STRIPE)],
                        d_hbm.at[c].at[pl.ds(row0, STRIPE)])

    return deg_k(dst3, z128, ones128)


def _sc_aggregate(h, src3, dst3, z128, m, na, nb):
    mesh = plsc.VectorSubcoreMesh(core_axis_name="c", subcore_axis_name="s")

    @functools.partial(
        pl.kernel,
        mesh=mesh,
        out_type=jax.ShapeDtypeStruct((NC, N_R, D), jnp.float32),
        scratch_types=[
            pltpu.VMEM((m, CHUNK), jnp.int32),
            pltpu.VMEM((m, CHUNK), jnp.int32),
            pltpu.VMEM((CHUNK, D), jnp.float32),
            pltpu.VMEM_SHARED((N_R, D), jnp.float32),
        ],
    )
    def agg(h_hbm, src_hbm, dst_hbm, z128_hbm, p_hbm,
            src_v, dst_v, rows_v, acc_s):
        c = lax.axis_index("c")
        s = lax.axis_index("s")
        wid = c * NS + s
        n_loc = jnp.where(c == 0, na, nb)
        pltpu.sync_copy(src_hbm.at[wid], src_v)
        pltpu.sync_copy(dst_hbm.at[wid], dst_v)
        row0 = s * STRIPE
        pltpu.sync_copy(z128_hbm, acc_s.at[pl.ds(row0, STRIPE)])
        plsc.subcore_barrier()

        def body(j, carry):
            pltpu.sync_copy(h_hbm.at[src_v.at[j]], rows_v)
            pltpu.sync_copy(rows_v, acc_s.at[dst_v.at[j]], add=True)
            return carry

        lax.fori_loop(0, n_loc, body, 0)
        plsc.subcore_barrier()
        pltpu.sync_copy(acc_s.at[pl.ds(row0, STRIPE)],
                        p_hbm.at[c].at[pl.ds(row0, STRIPE)])

    return agg(h, src3, dst3, z128)


BLK = 400


def _tc_layer(p, d, h, Wl, bl, Wr, relu):
    nb = N // BLK

    def body(p_ref, d_ref, h_ref, wl_ref, bl_ref, wr_ref, o_ref):
        deg = d_ref[0, :, 0:1] + d_ref[1, :, 0:1]
        mean = (p_ref[0] + p_ref[1]) / jnp.maximum(deg, 1.0)
        out = (lax.dot_general(mean, wl_ref[...], (((1,), (1,)), ((), ())),
                               preferred_element_type=jnp.float32)
               + lax.dot_general(h_ref[...], wr_ref[...],
                                 (((1,), (1,)), ((), ())),
                                 preferred_element_type=jnp.float32)
               + bl_ref[...])
        if relu:
            out = jnp.maximum(out, 0.0)
        o_ref[...] = out

    return pl.pallas_call(
        body,
        grid=(nb,),
        in_specs=[
            pl.BlockSpec((NC, BLK, D), lambda i: (0, i, 0)),
            pl.BlockSpec((NC, BLK, D), lambda i: (0, i, 0)),
            pl.BlockSpec((BLK, D), lambda i: (i, 0)),
            pl.BlockSpec((D, D), lambda i: (0, 0)),
            pl.BlockSpec((1, D), lambda i: (0, 0)),
            pl.BlockSpec((D, D), lambda i: (0, 0)),
        ],
        out_specs=pl.BlockSpec((BLK, D), lambda i: (i, 0)),
        out_shape=jax.ShapeDtypeStruct((N, D), jnp.float32),
    )(p, d, h, Wl, bl, Wr)


def kernel(x, edge_index, Wl1, bl1, Wr1, Wl2, bl2, Wr2, Wl3, bl3, Wr3):
    src = edge_index[0]
    dst = edge_index[1]
    e = src.shape[0]
    n_chunks = -(-e // (NS * CHUNK))
    na = (n_chunks * SPLIT_A + SPLIT_A + SPLIT_B - 1) // (SPLIT_A + SPLIT_B)
    nb = n_chunks - na
    if nb < 1:
        nb = 1
    m = max(na, nb)
    e_pad = NS * CHUNK * (na + nb)
    pad = e_pad - e
    sinkv = jnp.full((pad,), N, jnp.int32)
    src_p = jnp.concatenate([src, jnp.zeros((pad,), jnp.int32)])
    dst_p = jnp.concatenate([dst, sinkv])

    def part(v):
        p0 = v[:NS * na * CHUNK].reshape(NS, na, CHUNK)
        p1 = v[NS * na * CHUNK:].reshape(NS, nb, CHUNK)
        f0 = jnp.concatenate(
            [p0, jnp.full((NS, m - na, CHUNK), N, jnp.int32)], axis=1)
        f1 = jnp.concatenate(
            [p1, jnp.full((NS, m - nb, CHUNK), N, jnp.int32)], axis=1)
        return jnp.concatenate([f0, f1], axis=0)

    src_p = part(jnp.where(jnp.arange(e_pad) < e, src_p, 0))
    dst_p = part(dst_p)
    z128 = jnp.zeros((STRIPE, D), jnp.float32)
    ones128 = jnp.ones((CHUNK, D), jnp.float32)

    dg = _sc_degree(dst_p, z128, ones128, m, na, nb)[:, :N]

    def layer(h, Wl, bl, Wr, relu):
        p = _sc_aggregate(h, src_p, dst_p, z128, m, na, nb)
        return _tc_layer(p[:, :N], dg, h, Wl, bl.reshape(1, D), Wr, relu)

    h = layer(x, Wl1, bl1, Wr1, True)
    h = layer(h, Wl2, bl2, Wr2, True)
    return layer(h, Wl3, bl3, Wr3, False)
